# Initial kernel scaffold; baseline (speedup 1.0000x reference)
import jax
import jax.numpy as jnp
from jax.experimental import pallas as pl


def kernel(frame_ids, betas_w, global_orient_w, transl_w, body_pose_w):
    raise NotImplementedError("write your pallas kernel here")


# speedup vs baseline: 3.8185x; 3.8185x over previous
import functools

import jax
import jax.numpy as jnp
from jax import lax
from jax.experimental import pallas as pl
from jax.experimental.pallas import tpu as pltpu
from jax.experimental.pallas import tpu_sc as plsc

_B = 4096
_V = 100000
_NUM_CORES = 2
_NUM_SUBCORES = 16
_NW = _NUM_CORES * _NUM_SUBCORES
_D_BETAS = 10
_D_ORIENT = 3
_D_TRANSL = 3
_D_POSE = 69
_N_UNITS = _D_POSE + _D_ORIENT + _D_TRANSL + _D_BETAS
_ROUNDS = 3


def _lookup_body(ids_hbm, betas_hbm, orient_hbm, transl_hbm, pose_hbm,
                 betas_out, orient_out, transl_out, pose_out,
                 idx_v, row_v, out_v, bet_v):
    wid = lax.axis_index("s") * _NUM_CORES + lax.axis_index("c")

    pltpu.sync_copy(ids_hbm, idx_v)
    pltpu.sync_copy(betas_hbm.at[0], bet_v)

    def gather_units(k, _):
        idx16 = idx_v[pl.ds(k * 16, 16)]
        out_v[pl.ds(k * 16, 16)] = plsc.load_gather(row_v, [idx16])
        return _

    def splat_units(j):
        def body(k, _):
            vals = plsc.load_gather(bet_v, [jnp.full((16,), j, jnp.int32)])
            out_v[pl.ds(k * 16, 16)] = vals
            return _
        return body

    for r in range(_ROUNDS):
        u = r * _NW + wid

        @pl.when(u < _D_POSE)
        def _():
            pltpu.sync_copy(pose_hbm.at[u], row_v)
            lax.fori_loop(0, _B // 16, gather_units, 0)
            pltpu.sync_copy(out_v, pose_out.at[u])

        @pl.when((u >= _D_POSE) & (u < _D_POSE + _D_ORIENT))
        def _():
            j = u - _D_POSE
            pltpu.sync_copy(orient_hbm.at[j], row_v)
            lax.fori_loop(0, _B // 16, gather_units, 0)
            pltpu.sync_copy(out_v, orient_out.at[j])

        @pl.when((u >= _D_POSE + _D_ORIENT) & (u < _D_POSE + _D_ORIENT + _D_TRANSL))
        def _():
            j = u - (_D_POSE + _D_ORIENT)
            pltpu.sync_copy(transl_hbm.at[j], row_v)
            lax.fori_loop(0, _B // 16, gather_units, 0)
            pltpu.sync_copy(out_v, transl_out.at[j])

        @pl.when((u >= _D_POSE + _D_ORIENT + _D_TRANSL) & (u < _N_UNITS))
        def _():
            j = u - (_D_POSE + _D_ORIENT + _D_TRANSL)
            lax.fori_loop(0, _B // 16, splat_units(j), 0)
            pltpu.sync_copy(out_v, betas_out.at[j])


@jax.jit
def kernel(frame_ids, betas_w, global_orient_w, transl_w, body_pose_w):
    mesh = plsc.VectorSubcoreMesh(
        core_axis_name="c", subcore_axis_name="s",
        num_cores=_NUM_CORES, num_subcores=_NUM_SUBCORES)
    run = pl.kernel(
        _lookup_body,
        out_type=(
            jax.ShapeDtypeStruct((_D_BETAS, _B), jnp.float32),
            jax.ShapeDtypeStruct((_D_ORIENT, _B), jnp.float32),
            jax.ShapeDtypeStruct((_D_TRANSL, _B), jnp.float32),
            jax.ShapeDtypeStruct((_D_POSE, _B), jnp.float32),
        ),
        mesh=mesh,
        compiler_params=pltpu.CompilerParams(use_tc_tiling_on_sc=True, needs_layout_passes=False),
        scratch_types=[
            pltpu.VMEM((_B,), jnp.int32),
            pltpu.VMEM((_V,), jnp.float32),
            pltpu.VMEM((_B,), jnp.float32),
            pltpu.VMEM((10,), jnp.float32),
        ],
    )
    betas_t, orient_t, transl_t, pose_t = run(
        frame_ids.astype(jnp.int32), betas_w,
        jnp.swapaxes(global_orient_w, 0, 1),
        jnp.swapaxes(transl_w, 0, 1),
        jnp.swapaxes(body_pose_w, 0, 1))
    return (jnp.swapaxes(betas_t, 0, 1), jnp.swapaxes(orient_t, 0, 1),
            jnp.swapaxes(transl_t, 0, 1), jnp.swapaxes(pose_t, 0, 1))


# speedup vs baseline: 3.9101x; 1.0240x over previous
import functools

import jax
import jax.numpy as jnp
from jax import lax
from jax.experimental import pallas as pl
from jax.experimental.pallas import tpu as pltpu
from jax.experimental.pallas import tpu_sc as plsc

_B = 4096
_V = 100000
_NUM_CORES = 2
_NUM_SUBCORES = 16
_NW = _NUM_CORES * _NUM_SUBCORES
_D_BETAS = 10
_D_ORIENT = 3
_D_TRANSL = 3
_D_POSE = 69
_N_UNITS = _D_POSE + _D_ORIENT + _D_TRANSL + _D_BETAS
_ROUNDS = 3


def _lookup_body(ids_hbm, betas_hbm, orient_hbm, transl_hbm, pose_hbm,
                 betas_out, orient_out, transl_out, pose_out,
                 idx_v, row_v, out_v, bet_v):
    wid = lax.axis_index("s") * _NUM_CORES + lax.axis_index("c")

    pltpu.sync_copy(ids_hbm, idx_v)
    pltpu.sync_copy(betas_hbm.at[0], bet_v)

    _UNROLL = 8

    def gather_units(k, _):
        for s in range(_UNROLL):
            idx16 = idx_v[pl.ds(k * (16 * _UNROLL) + s * 16, 16)]
            out_v[pl.ds(k * (16 * _UNROLL) + s * 16, 16)] = (
                plsc.load_gather(row_v, [idx16]))
        return _

    def splat_units(j):
        def body(k, _):
            vals = plsc.load_gather(bet_v, [jnp.full((16,), j, jnp.int32)])
            for s in range(_UNROLL):
                out_v[pl.ds(k * (16 * _UNROLL) + s * 16, 16)] = vals
            return _
        return body

    for r in range(_ROUNDS):
        u = r * _NW + wid

        @pl.when(u < _D_POSE)
        def _():
            pltpu.sync_copy(pose_hbm.at[u], row_v)
            lax.fori_loop(0, _B // (16 * _UNROLL), gather_units, 0)
            pltpu.sync_copy(out_v, pose_out.at[u])

        @pl.when((u >= _D_POSE) & (u < _D_POSE + _D_ORIENT))
        def _():
            j = u - _D_POSE
            pltpu.sync_copy(orient_hbm.at[j], row_v)
            lax.fori_loop(0, _B // (16 * _UNROLL), gather_units, 0)
            pltpu.sync_copy(out_v, orient_out.at[j])

        @pl.when((u >= _D_POSE + _D_ORIENT) & (u < _D_POSE + _D_ORIENT + _D_TRANSL))
        def _():
            j = u - (_D_POSE + _D_ORIENT)
            pltpu.sync_copy(transl_hbm.at[j], row_v)
            lax.fori_loop(0, _B // (16 * _UNROLL), gather_units, 0)
            pltpu.sync_copy(out_v, transl_out.at[j])

        @pl.when((u >= _D_POSE + _D_ORIENT + _D_TRANSL) & (u < _N_UNITS))
        def _():
            j = u - (_D_POSE + _D_ORIENT + _D_TRANSL)
            lax.fori_loop(0, _B // (16 * _UNROLL), splat_units(j), 0)
            pltpu.sync_copy(out_v, betas_out.at[j])


@jax.jit
def kernel(frame_ids, betas_w, global_orient_w, transl_w, body_pose_w):
    mesh = plsc.VectorSubcoreMesh(
        core_axis_name="c", subcore_axis_name="s",
        num_cores=_NUM_CORES, num_subcores=_NUM_SUBCORES)
    run = pl.kernel(
        _lookup_body,
        out_type=(
            jax.ShapeDtypeStruct((_D_BETAS, _B), jnp.float32),
            jax.ShapeDtypeStruct((_D_ORIENT, _B), jnp.float32),
            jax.ShapeDtypeStruct((_D_TRANSL, _B), jnp.float32),
            jax.ShapeDtypeStruct((_D_POSE, _B), jnp.float32),
        ),
        mesh=mesh,
        compiler_params=pltpu.CompilerParams(use_tc_tiling_on_sc=True, needs_layout_passes=False),
        scratch_types=[
            pltpu.VMEM((_B,), jnp.int32),
            pltpu.VMEM((_V,), jnp.float32),
            pltpu.VMEM((_B,), jnp.float32),
            pltpu.VMEM((10,), jnp.float32),
        ],
    )
    betas_t, orient_t, transl_t, pose_t = run(
        frame_ids.astype(jnp.int32), betas_w,
        jnp.swapaxes(global_orient_w, 0, 1),
        jnp.swapaxes(transl_w, 0, 1),
        jnp.swapaxes(body_pose_w, 0, 1))
    return (jnp.swapaxes(betas_t, 0, 1), jnp.swapaxes(orient_t, 0, 1),
            jnp.swapaxes(transl_t, 0, 1), jnp.swapaxes(pose_t, 0, 1))
